# baseline (speedup 1.0000x reference)
import jax
import jax.numpy as jnp
from jax import lax
from jax.experimental import pallas as pl
from jax.experimental.pallas import tpu as pltpu
from jax.experimental.pallas import tpu_sc as plsc

NUM_CANVASES = 3
B, C, H, W = 16, 3, 512, 512

_ROWS = B * C * H
_NC, _NS = 2, 16
_NW = _NC * _NS
_ROWS_PER_W = _ROWS // _NW
_CH = 64
_NCH = _ROWS_PER_W // _CH
_NBUF = 3


def _sc_copy_body(src, dst, bufs, isems, osems):
    wid = lax.axis_index("s") * _NC + lax.axis_index("c")
    base = wid * _ROWS_PER_W

    def start_in(j):
        b = j % _NBUF
        return pltpu.async_copy(
            src.at[pl.ds(base + j * _CH, _CH)], bufs.at[b], isems.at[b]
        )

    def start_out(j):
        b = j % _NBUF
        return pltpu.async_copy(
            bufs.at[b], dst.at[pl.ds(base + j * _CH, _CH)], osems.at[b]
        )

    descs = {}
    for j in range(_NBUF):
        descs[("i", j)] = start_in(j)
    for i in range(_NCH):
        descs[("i", i)].wait()
        descs[("o", i)] = start_out(i)
        if i >= 1 and i + 2 < _NCH:
            descs[("o", i - 1)].wait()
            descs[("i", i + 2)] = start_in(i + 2)
    for i in range(_NCH - _NBUF, _NCH):
        descs[("o", i)].wait()


_sc_copy = pl.kernel(
    _sc_copy_body,
    out_type=jax.ShapeDtypeStruct((_ROWS, W), jnp.float32),
    mesh=plsc.VectorSubcoreMesh(
        core_axis_name="c", subcore_axis_name="s", num_cores=_NC, num_subcores=_NS
    ),
    scratch_types=[
        pltpu.VMEM((_NBUF, _CH, W), jnp.float32),
        pltpu.SemaphoreType.DMA((_NBUF,)),
        pltpu.SemaphoreType.DMA((_NBUF,)),
    ],
)


_TC_BLOCK = 6144


def _tc_copy_body(src_ref, dst_ref):
    dst_ref[...] = src_ref[...]


def _tc_copy(flat):
    return pl.pallas_call(
        _tc_copy_body,
        grid=(_ROWS // _TC_BLOCK,),
        in_specs=[pl.BlockSpec((_TC_BLOCK, W), lambda i: (i, 0))],
        out_specs=pl.BlockSpec((_TC_BLOCK, W), lambda i: (i, 0)),
        out_shape=jax.ShapeDtypeStruct((_ROWS, W), jnp.float32),
    )(flat)


def kernel(img_batch, canvases):
    del canvases
    flat = img_batch.reshape(_ROWS, W)
    a = _sc_copy(flat)
    b = _tc_copy(flat)
    return (a.reshape(B, C, H, W), b.reshape(B, C, H, W))


# speedup vs baseline: 1.5144x; 1.5144x over previous
import jax
import jax.numpy as jnp
from jax import lax
from jax.experimental import pallas as pl
from jax.experimental.pallas import tpu as pltpu
from jax.experimental.pallas import tpu_sc as plsc

NUM_CANVASES = 3
B, C, H, W = 16, 3, 512, 512

_ROWS = B * C * H
_NC, _NS = 2, 16
_NW = _NC * _NS
_ROWS_PER_W = _ROWS // _NW
_CH = 48
_NCH = _ROWS_PER_W // _CH
_NBUF = 4
_OLAG = 2


def _sc_copy_body(src, dst, bufs, isems, osems):
    wid = lax.axis_index("s") * _NC + lax.axis_index("c")
    base = wid * _ROWS_PER_W

    def start_in(j):
        b = j % _NBUF
        return pltpu.async_copy(
            src.at[pl.ds(base + j * _CH, _CH)], bufs.at[b], isems.at[b]
        )

    def start_out(j):
        b = j % _NBUF
        return pltpu.async_copy(
            bufs.at[b], dst.at[pl.ds(base + j * _CH, _CH)], osems.at[b]
        )

    descs = {}
    for j in range(_NBUF):
        descs[("i", j)] = start_in(j)
    for i in range(_NCH):
        descs[("i", i)].wait()
        descs[("o", i)] = start_out(i)
        j = i - _OLAG + _NBUF
        if i >= _OLAG and j < _NCH:
            descs[("o", i - _OLAG)].wait()
            descs[("i", j)] = start_in(j)
    for i in range(_NCH - _NBUF, _NCH):
        descs[("o", i)].wait()


_sc_copy = pl.kernel(
    _sc_copy_body,
    out_type=jax.ShapeDtypeStruct((_ROWS, W), jnp.float32),
    mesh=plsc.VectorSubcoreMesh(
        core_axis_name="c", subcore_axis_name="s", num_cores=_NC, num_subcores=_NS
    ),
    scratch_types=[
        pltpu.VMEM((_NBUF, _CH, W), jnp.float32),
        pltpu.SemaphoreType.DMA((_NBUF,)),
        pltpu.SemaphoreType.DMA((_NBUF,)),
    ],
)


def kernel(img_batch, canvases):
    del canvases
    flat = img_batch.reshape(_ROWS, W)
    return _sc_copy(flat).reshape(B, C, H, W)
